# baseline (speedup 1.0000x reference)
import jax
import jax.numpy as jnp
from jax.experimental import pallas as pl
from jax.experimental.pallas import tpu as pltpu

_BLOCK_ROWS = 1024


def _copy_out(x_ref, o_hbm, sem):
    i = pl.program_id(0)
    dst = o_hbm.at[pl.ds(i * _BLOCK_ROWS, _BLOCK_ROWS), :]
    copy = pltpu.make_async_copy(x_ref, dst, sem)
    copy.start()
    copy.wait()


def kernel(hidden_states, routing_weights, selected_experts):
    del routing_weights, selected_experts
    tokens, d_model = hidden_states.shape
    return pl.pallas_call(
        _copy_out,
        grid=(tokens // _BLOCK_ROWS,),
        in_specs=[pl.BlockSpec((_BLOCK_ROWS, d_model), lambda i: (i, 0))],
        out_specs=pl.BlockSpec(memory_space=pl.ANY),
        out_shape=jax.ShapeDtypeStruct((tokens, d_model), hidden_states.dtype),
        scratch_shapes=[pltpu.SemaphoreType.DMA],
        compiler_params=pltpu.CompilerParams(dimension_semantics=("arbitrary",)),
    )(hidden_states)


# speedup vs baseline: 1.0100x; 1.0100x over previous
import jax
import jax.numpy as jnp
from jax.experimental import pallas as pl
from jax.experimental.pallas import tpu as pltpu


def _copy_block(x_ref, o_ref):
    o_ref[...] = x_ref[...]


def kernel(hidden_states, routing_weights, selected_experts):
    del routing_weights, selected_experts
    tokens, d_model = hidden_states.shape
    block_rows = 1016
    return pl.pallas_call(
        _copy_block,
        grid=(pl.cdiv(tokens, block_rows),),
        in_specs=[pl.BlockSpec((block_rows, d_model), lambda i: (i, 0))],
        out_specs=pl.BlockSpec((block_rows, d_model), lambda i: (i, 0)),
        out_shape=jax.ShapeDtypeStruct((tokens, d_model), hidden_states.dtype),
        compiler_params=pltpu.CompilerParams(dimension_semantics=("parallel",), vmem_limit_bytes=134217728),
    )(hidden_states)
